# baseline (speedup 1.0000x reference)
import functools

import jax
import jax.numpy as jnp
from jax import lax
from jax.experimental import pallas as pl
from jax.experimental.pallas import tpu as pltpu
from jax.experimental.pallas import tpu_sc as plsc

_N = 10000
_E = 320000
_D = 128
_NPAD = 10240
_NTILES = 32
_EPT = _E // _NTILES
_CHUNK = 96
_BLK = 12
_NBLK = 9
_NCHUNK = _BLK * _NBLK
_EPT_PAD = _NCHUNK * _CHUNK
_EPAD = _NTILES * _EPT_PAD
_ROWS_PER_TILE = _NPAD // 16


def _mesh():
    return plsc.VectorSubcoreMesh(core_axis_name="c", subcore_axis_name="s")


def _sc_params():
    return pltpu.CompilerParams(needs_layout_passes=False)


@functools.lru_cache(maxsize=None)
def _deg_kernel():
    @functools.partial(
        pl.kernel,
        out_type=jax.ShapeDtypeStruct((_NTILES, 2, _NPAD), jnp.float32),
        mesh=_mesh(),
        compiler_params=_sc_params(),
        scratch_types=[
            pltpu.VMEM((_EPT,), jnp.int32),
            pltpu.VMEM((_EPT,), jnp.int32),
            pltpu.VMEM((_NPAD,), jnp.float32),
            pltpu.VMEM((_NPAD,), jnp.float32),
        ],
    )
    def deg(src_hbm, dst_hbm, out_hbm, src_v, dst_v, tsrc_v, tdst_v):
        c = lax.axis_index("c")
        s = lax.axis_index("s")
        wid = c * 16 + s
        zero16 = jnp.zeros((16,), jnp.float32)

        def zero_body(i, carry):
            tsrc_v[pl.ds(i * 16, 16)] = zero16
            tdst_v[pl.ds(i * 16, 16)] = zero16
            return carry

        lax.fori_loop(0, _NPAD // 16, zero_body, 0)

        pltpu.sync_copy(src_hbm.at[pl.ds(wid * _EPT, _EPT)], src_v)
        pltpu.sync_copy(dst_hbm.at[pl.ds(wid * _EPT, _EPT)], dst_v)

        ones16 = jnp.ones((16,), jnp.float32)

        def count_body(i, carry):
            si = src_v[pl.ds(i * 16, 16)]
            di = dst_v[pl.ds(i * 16, 16)]
            plsc.addupdate_scatter(tsrc_v, [si], ones16)
            plsc.addupdate_scatter(tdst_v, [di], ones16)
            return carry

        lax.fori_loop(0, _EPT // 16, count_body, 0)

        pltpu.sync_copy(tsrc_v, out_hbm.at[wid, 0])
        pltpu.sync_copy(tdst_v, out_hbm.at[wid, 1])

    return deg


@functools.lru_cache(maxsize=None)
def _agg_kernel():
    @functools.partial(
        pl.kernel,
        out_type=jax.ShapeDtypeStruct((2, _NPAD, _D), jnp.float32),
        mesh=_mesh(),
        compiler_params=_sc_params(),
        scratch_types=[
            pltpu.VMEM((2, _BLK, _CHUNK), jnp.int32),
            pltpu.VMEM((2, _BLK, _CHUNK), jnp.int32),
            pltpu.VMEM((_CHUNK, _D), jnp.float32),
            pltpu.VMEM((_CHUNK, _D), jnp.float32),
            pltpu.VMEM((_CHUNK, _D), jnp.float32),
            pltpu.VMEM_SHARED((_NPAD, _D), jnp.float32),
            pltpu.SemaphoreType.DMA,
            pltpu.SemaphoreType.DMA,
            pltpu.SemaphoreType.DMA,
        ],
    )
    def agg(h_hbm, src_hbm, dst_hbm, zeros_hbm, out_hbm,
            sidx, didx, rows_a, rows_b, rows_c, acc_sh, gsem, ssem, isem):
        c = lax.axis_index("c")
        s = lax.axis_index("s")
        wid = c * 16 + s
        r0 = s * _ROWS_PER_TILE
        bufs = (rows_a, rows_b, rows_c)

        pltpu.sync_copy(zeros_hbm.at[pl.ds(r0, _ROWS_PER_TILE)],
                        acc_sh.at[pl.ds(r0, _ROWS_PER_TILE)])
        pltpu.sync_copy(src_hbm.at[wid, 0], sidx.at[0])
        pltpu.sync_copy(dst_hbm.at[wid, 0], didx.at[0])
        pltpu.sync_copy(zeros_hbm.at[pl.ds(0, _CHUNK)], rows_c)
        plsc.subcore_barrier()

        def gath(idx_row, buf):
            pltpu.async_copy(h_hbm.at[idx_row], buf, gsem)

        def scat(idx_row, buf):
            pltpu.async_copy(buf, acc_sh.at[idx_row], ssem, add=True)

        def drain_rows(buf, sem):
            pltpu.make_async_copy(h_hbm.at[pl.ds(0, _CHUNK)], buf,
                                  sem).wait()

        def drain_idx():
            pltpu.make_async_copy(src_hbm.at[wid, 0], sidx.at[0],
                                  isem).wait()

        gath(sidx.at[0, 0], rows_a)
        gath(sidx.at[0, 1], rows_b)
        scat(didx.at[0, 0], rows_c)

        def body(k, carry):
            p = lax.rem(k, 2)
            np_ = 1 - p
            pltpu.async_copy(src_hbm.at[wid, k + 1], sidx.at[np_], isem)
            pltpu.async_copy(dst_hbm.at[wid, k + 1], didx.at[np_], isem)
            for j in range(_BLK):
                if j == _BLK - 2:
                    drain_idx()
                    drain_idx()
                buf = bufs[j % 3]
                nbuf = bufs[(j + 2) % 3]
                drain_rows(buf, gsem)
                drain_rows(nbuf, ssem)
                if j < _BLK - 2:
                    gath(sidx.at[p, j + 2], nbuf)
                else:
                    gath(sidx.at[np_, j + 2 - _BLK], nbuf)
                scat(didx.at[p, j], buf)
            return carry

        lax.fori_loop(0, _NBLK - 1, body, 0, unroll=False)

        lp = (_NBLK - 1) % 2
        for j in range(_BLK):
            buf = bufs[j % 3]
            nbuf = bufs[(j + 2) % 3]
            drain_rows(buf, gsem)
            drain_rows(nbuf, ssem)
            if j < _BLK - 2:
                gath(sidx.at[lp, j + 2], nbuf)
            scat(didx.at[lp, j], buf)
        drain_rows(bufs[(_BLK - 1) % 3], ssem)

        plsc.subcore_barrier()
        pltpu.sync_copy(acc_sh.at[pl.ds(r0, _ROWS_PER_TILE)],
                        out_hbm.at[c, pl.ds(r0, _ROWS_PER_TILE)])

    return agg


_ROWS_BLK = 2048


def _norms(deg_ref):
    d = deg_ref[...]
    out_deg = jnp.sum(d[:, :_NTILES], axis=1, keepdims=True)
    in_deg = jnp.sum(d[:, _NTILES:], axis=1, keepdims=True)
    ns = lax.rsqrt(jnp.maximum(out_deg, 1.0))
    nd = lax.rsqrt(jnp.maximum(in_deg, 1.0))
    return ns, nd


def _tc1a_body(x_ref, w_ref, o_ref):
    o_ref[...] = jnp.dot(x_ref[...], w_ref[...],
                         preferred_element_type=jnp.float32)


def _tc1b_body(deg_ref, h_ref, o_ref):
    ns, _ = _norms(deg_ref)
    o_ref[...] = h_ref[...] * ns


def _tc2_body(deg_ref, a0_ref, a1_ref, b_ref, w_ref, o_ref):
    ns, nd = _norms(deg_ref)
    h = (a0_ref[...] + a1_ref[...]) * nd + b_ref[...]
    h = jnp.dot(h, w_ref[...], preferred_element_type=jnp.float32)
    o_ref[...] = h * ns


def _tc3_body(deg_ref, a0_ref, a1_ref, b_ref, o_ref):
    _, nd = _norms(deg_ref)
    o_ref[...] = (a0_ref[...] + a1_ref[...]) * nd + b_ref[...]


_GRID = _NPAD // _ROWS_BLK

_DEG_SPEC = pl.BlockSpec((_ROWS_BLK, 2 * _NTILES), lambda i: (i, 0))
_MAT_SPEC = pl.BlockSpec((_ROWS_BLK, _D), lambda i: (i, 0))
_W_SPEC = pl.BlockSpec((_D, _D), lambda i: (0, 0))
_B_SPEC = pl.BlockSpec((1, _D), lambda i: (0, 0))
_OUT_TYPE = jax.ShapeDtypeStruct((_NPAD, _D), jnp.float32)


def _tc1a(x, w):
    return pl.pallas_call(
        _tc1a_body, grid=(_GRID,),
        in_specs=[_MAT_SPEC, _W_SPEC],
        out_specs=_MAT_SPEC, out_shape=_OUT_TYPE,
    )(x, w)


def _tc1b(deg, h):
    return pl.pallas_call(
        _tc1b_body, grid=(_GRID,),
        in_specs=[_DEG_SPEC, _MAT_SPEC],
        out_specs=_MAT_SPEC, out_shape=_OUT_TYPE,
    )(deg, h)


def _tc2(deg, a0, a1, b, w):
    return pl.pallas_call(
        _tc2_body, grid=(_GRID,),
        in_specs=[_DEG_SPEC, _MAT_SPEC, _MAT_SPEC, _B_SPEC, _W_SPEC],
        out_specs=_MAT_SPEC, out_shape=_OUT_TYPE,
    )(deg, a0, a1, b, w)


def _tc3(deg, a0, a1, b):
    return pl.pallas_call(
        _tc3_body, grid=(_GRID,),
        in_specs=[_DEG_SPEC, _MAT_SPEC, _MAT_SPEC, _B_SPEC],
        out_specs=_MAT_SPEC, out_shape=_OUT_TYPE,
    )(deg, a0, a1, b)


def kernel(in_feat, edge_index, W0, b0, W1, b1):
    ei = edge_index.astype(jnp.int32)
    src = ei[0]
    dst = ei[1]
    npadrow = _EPAD - _E
    pad_idx = _N + jnp.arange(npadrow, dtype=jnp.int32) % (_NPAD - _N)
    epad = jnp.stack([pad_idx, pad_idx])
    eip = jnp.concatenate([ei, epad], axis=1)
    src4 = eip[0].reshape(_NTILES, _NBLK, _BLK, _CHUNK)
    dst4 = eip[1].reshape(_NTILES, _NBLK, _BLK, _CHUNK)

    xp = jnp.zeros((_NPAD, _D), jnp.float32).at[:_N].set(in_feat)
    zeros = jnp.zeros((_NPAD, _D), jnp.float32)
    b0r = b0.reshape(1, _D)
    b1r = b1.reshape(1, _D)

    degp = _deg_kernel()(src, dst)
    deg64 = degp.transpose(1, 0, 2).reshape(2 * _NTILES, _NPAD).T

    h1 = _tc1a(xp, W0)
    h1s = _tc1b(deg64, h1)
    m1 = _agg_kernel()(h1s, src4, dst4, zeros)
    h2s = _tc2(deg64, m1[0], m1[1], b0r, W1)
    m2 = _agg_kernel()(h2s, src4, dst4, zeros)
    out = _tc3(deg64, m2[0], m2[1], b1r)
    return out[:_N]


# speedup vs baseline: 1.0381x; 1.0381x over previous
import functools

import jax
import jax.numpy as jnp
from jax import lax
from jax.experimental import pallas as pl
from jax.experimental.pallas import tpu as pltpu
from jax.experimental.pallas import tpu_sc as plsc

_N = 10000
_E = 320000
_D = 128
_NPAD = 10240
_NTILES = 32
_EPT = _E // _NTILES
_CHUNK = 96
_BLK = 15
_NBLK = 7
_NCHUNK = _BLK * _NBLK
_EPT_PAD = _NCHUNK * _CHUNK
_EPAD = _NTILES * _EPT_PAD
_ROWS_PER_TILE = _NPAD // 16


def _mesh():
    return plsc.VectorSubcoreMesh(core_axis_name="c", subcore_axis_name="s")


def _sc_params():
    return pltpu.CompilerParams(needs_layout_passes=False)


@functools.lru_cache(maxsize=None)
def _deg_kernel():
    @functools.partial(
        pl.kernel,
        out_type=jax.ShapeDtypeStruct((_NTILES, 2, _NPAD), jnp.float32),
        mesh=_mesh(),
        compiler_params=_sc_params(),
        scratch_types=[
            pltpu.VMEM((_EPT,), jnp.int32),
            pltpu.VMEM((_EPT,), jnp.int32),
            pltpu.VMEM((_NPAD,), jnp.float32),
            pltpu.VMEM((_NPAD,), jnp.float32),
        ],
    )
    def deg(src_hbm, dst_hbm, out_hbm, src_v, dst_v, tsrc_v, tdst_v):
        c = lax.axis_index("c")
        s = lax.axis_index("s")
        wid = c * 16 + s
        zero16 = jnp.zeros((16,), jnp.float32)

        def zero_body(i, carry):
            tsrc_v[pl.ds(i * 16, 16)] = zero16
            tdst_v[pl.ds(i * 16, 16)] = zero16
            return carry

        lax.fori_loop(0, _NPAD // 16, zero_body, 0)

        pltpu.sync_copy(src_hbm.at[pl.ds(wid * _EPT, _EPT)], src_v)
        pltpu.sync_copy(dst_hbm.at[pl.ds(wid * _EPT, _EPT)], dst_v)

        ones16 = jnp.ones((16,), jnp.float32)

        def count_body(i, carry):
            si = src_v[pl.ds(i * 16, 16)]
            di = dst_v[pl.ds(i * 16, 16)]
            plsc.addupdate_scatter(tsrc_v, [si], ones16)
            plsc.addupdate_scatter(tdst_v, [di], ones16)
            return carry

        lax.fori_loop(0, _EPT // 16, count_body, 0)

        pltpu.sync_copy(tsrc_v, out_hbm.at[wid, 0])
        pltpu.sync_copy(tdst_v, out_hbm.at[wid, 1])

    return deg


@functools.lru_cache(maxsize=None)
def _agg_kernel():
    @functools.partial(
        pl.kernel,
        out_type=jax.ShapeDtypeStruct((2, _NPAD, _D), jnp.float32),
        mesh=_mesh(),
        compiler_params=_sc_params(),
        scratch_types=[
            pltpu.VMEM((2, _BLK, _CHUNK), jnp.int32),
            pltpu.VMEM((2, _BLK, _CHUNK), jnp.int32),
            pltpu.VMEM((_CHUNK, _D), jnp.float32),
            pltpu.VMEM((_CHUNK, _D), jnp.float32),
            pltpu.VMEM((_CHUNK, _D), jnp.float32),
            pltpu.VMEM_SHARED((_NPAD, _D), jnp.float32),
            pltpu.SemaphoreType.DMA,
            pltpu.SemaphoreType.DMA,
            pltpu.SemaphoreType.DMA,
        ],
    )
    def agg(h_hbm, src_hbm, dst_hbm, zeros_hbm, out_hbm,
            sidx, didx, rows_a, rows_b, rows_c, acc_sh, gsem, ssem, isem):
        c = lax.axis_index("c")
        s = lax.axis_index("s")
        wid = c * 16 + s
        r0 = s * _ROWS_PER_TILE
        bufs = (rows_a, rows_b, rows_c)

        pltpu.sync_copy(zeros_hbm.at[pl.ds(r0, _ROWS_PER_TILE)],
                        acc_sh.at[pl.ds(r0, _ROWS_PER_TILE)])
        pltpu.sync_copy(src_hbm.at[wid, 0], sidx.at[0])
        pltpu.sync_copy(dst_hbm.at[wid, 0], didx.at[0])
        pltpu.sync_copy(zeros_hbm.at[pl.ds(0, _CHUNK)], rows_c)
        plsc.subcore_barrier()

        def gath(idx_row, buf):
            pltpu.async_copy(h_hbm.at[idx_row], buf, gsem)

        def scat(idx_row, buf):
            pltpu.async_copy(buf, acc_sh.at[idx_row], ssem, add=True)

        def drain_rows(buf, sem):
            pltpu.make_async_copy(h_hbm.at[pl.ds(0, _CHUNK)], buf,
                                  sem).wait()

        def drain_idx():
            pltpu.make_async_copy(src_hbm.at[wid, 0], sidx.at[0],
                                  isem).wait()

        gath(sidx.at[0, 0], rows_a)
        gath(sidx.at[0, 1], rows_b)
        scat(didx.at[0, 0], rows_c)

        def body(k, carry):
            p = lax.rem(k, 2)
            np_ = 1 - p
            pltpu.async_copy(src_hbm.at[wid, k + 1], sidx.at[np_], isem)
            pltpu.async_copy(dst_hbm.at[wid, k + 1], didx.at[np_], isem)
            for j in range(_BLK):
                if j == _BLK - 2:
                    drain_idx()
                    drain_idx()
                buf = bufs[j % 3]
                nbuf = bufs[(j + 2) % 3]
                drain_rows(buf, gsem)
                drain_rows(nbuf, ssem)
                if j < _BLK - 2:
                    gath(sidx.at[p, j + 2], nbuf)
                else:
                    gath(sidx.at[np_, j + 2 - _BLK], nbuf)
                scat(didx.at[p, j], buf)
            return carry

        lax.fori_loop(0, _NBLK - 1, body, 0, unroll=False)

        lp = (_NBLK - 1) % 2
        for j in range(_BLK):
            buf = bufs[j % 3]
            nbuf = bufs[(j + 2) % 3]
            drain_rows(buf, gsem)
            drain_rows(nbuf, ssem)
            if j < _BLK - 2:
                gath(sidx.at[lp, j + 2], nbuf)
            scat(didx.at[lp, j], buf)
        drain_rows(bufs[(_BLK - 1) % 3], ssem)

        plsc.subcore_barrier()
        pltpu.sync_copy(acc_sh.at[pl.ds(r0, _ROWS_PER_TILE)],
                        out_hbm.at[c, pl.ds(r0, _ROWS_PER_TILE)])

    return agg


_ROWS_BLK = 2048


def _norms(deg_ref):
    d = deg_ref[...]
    out_deg = jnp.sum(d[:, :_NTILES], axis=1, keepdims=True)
    in_deg = jnp.sum(d[:, _NTILES:], axis=1, keepdims=True)
    ns = lax.rsqrt(jnp.maximum(out_deg, 1.0))
    nd = lax.rsqrt(jnp.maximum(in_deg, 1.0))
    return ns, nd


def _tc1_body(deg_ref, x_ref, w_ref, o_ref):
    ns, _ = _norms(deg_ref)
    h = jnp.dot(x_ref[...], w_ref[...], preferred_element_type=jnp.float32)
    o_ref[...] = h * ns


def _tc2_body(deg_ref, a0_ref, a1_ref, b_ref, w_ref, o_ref):
    ns, nd = _norms(deg_ref)
    h = (a0_ref[...] + a1_ref[...]) * nd + b_ref[...]
    h = jnp.dot(h, w_ref[...], preferred_element_type=jnp.float32)
    o_ref[...] = h * ns


def _tc3_body(deg_ref, a0_ref, a1_ref, b_ref, o_ref):
    _, nd = _norms(deg_ref)
    o_ref[...] = (a0_ref[...] + a1_ref[...]) * nd + b_ref[...]


_GRID = _NPAD // _ROWS_BLK

_DEG_SPEC = pl.BlockSpec((_ROWS_BLK, 2 * _NTILES), lambda i: (i, 0))
_MAT_SPEC = pl.BlockSpec((_ROWS_BLK, _D), lambda i: (i, 0))
_W_SPEC = pl.BlockSpec((_D, _D), lambda i: (0, 0))
_B_SPEC = pl.BlockSpec((1, _D), lambda i: (0, 0))
_OUT_TYPE = jax.ShapeDtypeStruct((_NPAD, _D), jnp.float32)


def _tc1(deg, x, w):
    return pl.pallas_call(
        _tc1_body, grid=(_GRID,),
        in_specs=[_DEG_SPEC, _MAT_SPEC, _W_SPEC],
        out_specs=_MAT_SPEC, out_shape=_OUT_TYPE,
    )(deg, x, w)


def _tc2(deg, a0, a1, b, w):
    return pl.pallas_call(
        _tc2_body, grid=(_GRID,),
        in_specs=[_DEG_SPEC, _MAT_SPEC, _MAT_SPEC, _B_SPEC, _W_SPEC],
        out_specs=_MAT_SPEC, out_shape=_OUT_TYPE,
    )(deg, a0, a1, b, w)


def _tc3(deg, a0, a1, b):
    return pl.pallas_call(
        _tc3_body, grid=(_GRID,),
        in_specs=[_DEG_SPEC, _MAT_SPEC, _MAT_SPEC, _B_SPEC],
        out_specs=_MAT_SPEC, out_shape=_OUT_TYPE,
    )(deg, a0, a1, b)


def kernel(in_feat, edge_index, W0, b0, W1, b1):
    ei = edge_index.astype(jnp.int32)
    src = ei[0]
    dst = ei[1]
    npadrow = _EPAD - _E
    pad_idx = _N + jnp.arange(npadrow, dtype=jnp.int32) % (_NPAD - _N)
    epad = jnp.stack([pad_idx, pad_idx])
    eip = jnp.concatenate([ei, epad], axis=1)
    src4 = eip[0].reshape(_NTILES, _NBLK, _BLK, _CHUNK)
    dst4 = eip[1].reshape(_NTILES, _NBLK, _BLK, _CHUNK)

    xp = jnp.zeros((_NPAD, _D), jnp.float32).at[:_N].set(in_feat)
    zeros = jnp.zeros((_NPAD, _D), jnp.float32)
    b0r = b0.reshape(1, _D)
    b1r = b1.reshape(1, _D)

    degp = _deg_kernel()(src, dst)
    deg64 = degp.transpose(1, 0, 2).reshape(2 * _NTILES, _NPAD).T

    h1s = _tc1(deg64, xp, W0)
    m1 = _agg_kernel()(h1s, src4, dst4, zeros)
    h2s = _tc2(deg64, m1[0], m1[1], b0r, W1)
    m2 = _agg_kernel()(h2s, src4, dst4, zeros)
    out = _tc3(deg64, m2[0], m2[1], b1r)
    return out[:_N]
